# baseline (speedup 1.0000x reference)
import functools

import jax
import jax.numpy as jnp
from jax import lax
from jax.experimental import pallas as pl
from jax.experimental.pallas import tpu as pltpu
from jax.experimental.pallas import tpu_sc as plsc

D_MODEL = 64
SCALE = 8.0

_INFO = plsc.get_sparse_core_info()
NC = _INFO.num_cores
NS = _INFO.num_subcores
NW = NC * NS
LANES = _INFO.num_lanes

TOK = 128
NG = 3
LEAD = 2
NO = 2


def _make_kernel(b: int, s: int):
  assert b == NW * TOK and s > NG

  mesh = plsc.VectorSubcoreMesh(core_axis_name="c", subcore_axis_name="s")

  @functools.partial(
      pl.kernel,
      out_type=jax.ShapeDtypeStruct((s, 8, NW, 8, TOK), jnp.float32),
      mesh=mesh,
      scratch_types=[
          pltpu.VMEM((s, TOK), jnp.int32),
          pltpu.VMEM((NG + 1, TOK), jnp.int32),
          pltpu.VMEM((NG + 1, TOK), jnp.int32),
          pltpu.VMEM((NG, TOK, TOK), jnp.float32),
          pltpu.VMEM((NO, D_MODEL, TOK), jnp.float32),
          pltpu.SemaphoreType.DMA,
          pltpu.SemaphoreType.DMA,
      ],
      compiler_params=pltpu.CompilerParams(needs_layout_passes=False),
  )
  def emb_kernel(idx_hbm, tbl_hbm, out_hbm, idx_v, row_v, base_v, g_v, ot_v,
                 gsem, psem):
    wid = lax.axis_index("s") * NC + lax.axis_index("c")
    pltpu.sync_copy(idx_hbm.at[wid], idx_v)

    def prep_and_fire(j):
      m = j % (NG + 1)
      for c in range(TOK // LANES):
        sl = pl.ds(c * LANES, LANES)
        v = idx_v[j, sl]
        row_v[m, sl] = lax.shift_right_logical(v, 1)
        base_v[m, sl] = lax.bitwise_and(v, 1) * D_MODEL
      pltpu.async_copy(tbl_hbm.at[row_v.at[m]], g_v.at[j % NG], gsem)

    def wait_gather():
      pltpu.make_async_copy(tbl_hbm.at[pl.ds(0, TOK)], g_v.at[0], gsem).wait()

    def wait_put():
      pltpu.make_async_copy(ot_v.at[0, pl.ds(0, 8)], out_hbm.at[0, 0, 0],
                            psem).wait()

    for j in range(LEAD):
      prep_and_fire(j)

    ones = jnp.ones((LANES,), jnp.int32)
    toks = [jnp.arange(c * LANES, (c + 1) * LANES, dtype=jnp.int32)
            for c in range(TOK // LANES)]

    @pl.loop(0, s)
    def _task(j):
      wait_gather()

      @pl.when(j + LEAD < s)
      def _():
        prep_and_fire(j + LEAD)

      @pl.when(j >= NO)
      def _():
        for _ in range(8):
          wait_put()

      m = j % (NG + 1)
      o = j % NO
      gb = g_v.at[j % NG]

      for c in range(TOK // LANES):
        sl = pl.ds(c * LANES, LANES)
        base_c = base_v[m, sl]

        tok = toks[c]

        @pl.loop(0, D_MODEL, init_carry=base_c, unroll=4)
        def _d(dd, dloc):
          ot_v[o, dd, sl] = plsc.load_gather(gb, [tok, dloc]) * SCALE
          return dloc + ones

      for g in range(8):
        pltpu.async_copy(
            ot_v.at[o, pl.ds(8 * g, 8)], out_hbm.at[j, g, wid], psem
        )

    @pl.loop(0, NO * 8)
    def _drain(_):
      wait_put()

  return emb_kernel


def kernel(x, table):
  b, s = x.shape
  idx3 = x.reshape(NW, TOK, s).transpose(0, 2, 1).astype(jnp.int32)
  tbl2 = table.reshape(table.shape[0] // 2, 128)
  out5 = _make_kernel(b, s)(idx3, tbl2)
  return jnp.transpose(out5, (2, 4, 0, 1, 3)).reshape(b, s, D_MODEL)


# speedup vs baseline: 1.0779x; 1.0779x over previous
import functools

import jax
import jax.numpy as jnp
import numpy as np
from jax import lax
from jax.experimental import pallas as pl
from jax.experimental.pallas import tpu as pltpu
from jax.experimental.pallas import tpu_sc as plsc

D_MODEL = 64
SCALE = 8.0

_INFO = plsc.get_sparse_core_info()
NC = _INFO.num_cores
NS = _INFO.num_subcores
NW = NC * NS
LANES = _INFO.num_lanes

TOK = 128
NG = 4
LEAD = 3
NO = 2


def _make_kernel(b: int, s: int):
  assert b == NW * TOK and s > NG

  mesh = plsc.VectorSubcoreMesh(core_axis_name="c", subcore_axis_name="s")

  @functools.partial(
      pl.kernel,
      out_type=jax.ShapeDtypeStruct((s, 8, NW, 8, TOK), jnp.float32),
      mesh=mesh,
      scratch_types=[
          pltpu.VMEM((s, TOK), jnp.int32),
          pltpu.VMEM((NG, TOK, D_MODEL), jnp.float32),
          pltpu.VMEM((NO, D_MODEL, TOK), jnp.float32),
          pltpu.VMEM((2, LANES, LANES), jnp.int32),
          pltpu.SemaphoreType.DMA,
          pltpu.SemaphoreType.DMA,
      ],
      compiler_params=pltpu.CompilerParams(use_tc_tiling_on_sc=False,
                                           needs_layout_passes=False),
  )
  def emb_kernel(idx_hbm, tbl_hbm, out_hbm, idx_v, g_v, ot_v, diag_v,
                 gsem, psem):
    wid = lax.axis_index("s") * NC + lax.axis_index("c")
    pltpu.sync_copy(idx_hbm.at[wid], idx_v)

    lane = lax.iota(jnp.int32, LANES)
    for k in range(LANES):
      diag_v[0, k] = lax.rem(lane + k, LANES)
      diag_v[1, k] = lane

    def fire_gather(j):
      pltpu.async_copy(tbl_hbm.at[idx_v.at[j]], g_v.at[j % NG], gsem)

    def wait_gather():
      pltpu.make_async_copy(tbl_hbm.at[pl.ds(0, TOK)], g_v.at[0], gsem).wait()

    def wait_put():
      pltpu.make_async_copy(ot_v.at[0, pl.ds(0, 8)],
                            out_hbm.at[0, 0, 0], psem).wait()

    for j in range(LEAD):
      fire_gather(j)

    @pl.loop(0, s)
    def _task(j):
      wait_gather()

      @pl.when(j + LEAD < s)
      def _():
        fire_gather(j + LEAD)

      @pl.when(j >= NO)
      def _():
        for _ in range(8):
          wait_put()

      o = j % NO
      gb = g_v.at[j % NG]
      ob = ot_v.at[o]

      @pl.loop(0, TOK // LANES)
      def _c(c):
        tok = lane + c * LANES
        for q in range(D_MODEL // LANES):
          for k in range(LANES):
            dq = diag_v[0, k] + (q * LANES)
            val = plsc.load_gather(gb, [tok, dq]) * SCALE
            plsc.store_scatter(ob, [dq, tok], val)

      for g in range(8):
        pltpu.async_copy(
            ot_v.at[o, pl.ds(g * 8, 8)], out_hbm.at[j, g, wid], psem
        )

    @pl.loop(0, NO * 8)
    def _drain(_):
      wait_put()

  return emb_kernel


def kernel(x, table):
  b, s = x.shape
  idx3 = x.reshape(NW, TOK, s).transpose(0, 2, 1).astype(jnp.int32)
  out5 = _make_kernel(b, s)(idx3, table)
  return jnp.transpose(out5, (2, 4, 0, 1, 3)).reshape(b, s, D_MODEL)


# speedup vs baseline: 1.3060x; 1.2116x over previous
import functools

import jax
import jax.numpy as jnp
from jax import lax
from jax.experimental import pallas as pl
from jax.experimental.pallas import tpu as pltpu
from jax.experimental.pallas import tpu_sc as plsc

D_MODEL = 64
SCALE = 8.0

_INFO = plsc.get_sparse_core_info()
NC = _INFO.num_cores
NS = _INFO.num_subcores
NW = NC * NS
LANES = _INFO.num_lanes

CHUNK = 128
NBUF = 4
HALF = NBUF // 2
NO = 2


def _make_kernel(n_idx: int):
  assert n_idx % (NW * CHUNK) == 0
  per_w = n_idx // NW
  n_chunks = per_w // CHUNK
  assert n_chunks > NBUF

  mesh = plsc.VectorSubcoreMesh(core_axis_name="c", subcore_axis_name="s")

  @functools.partial(
      pl.kernel,
      out_type=jax.ShapeDtypeStruct((n_idx, D_MODEL), jnp.float32),
      mesh=mesh,
      scratch_types=[
          pltpu.VMEM((n_chunks, CHUNK), jnp.int32),
          pltpu.VMEM((NBUF, CHUNK, 2 * D_MODEL), jnp.float32),
          pltpu.VMEM((NO, CHUNK, D_MODEL), jnp.float32),
          pltpu.SemaphoreType.DMA,
          pltpu.SemaphoreType.DMA,
      ],
      compiler_params=pltpu.CompilerParams(use_tc_tiling_on_sc=False),
  )
  def emb_kernel(idx_hbm, table_hbm, out_hbm, idx_v, rows_v, ot_v, gsem,
                 psem):
    wid = lax.axis_index("s") * NC + lax.axis_index("c")
    base = wid * per_w
    pltpu.sync_copy(idx_hbm.at[wid], idx_v)

    def fire_gather(j):
      pltpu.async_copy(table_hbm.at[idx_v.at[j]], rows_v.at[j % NBUF], gsem)

    def wait_gather():
      pltpu.make_async_copy(table_hbm.at[pl.ds(0, CHUNK)], rows_v.at[0],
                            gsem).wait()

    def wait_put():
      pltpu.make_async_copy(ot_v.at[0], out_hbm.at[pl.ds(0, CHUNK)],
                            psem).wait()

    for j in range(HALF):
      fire_gather(j)

    @pl.loop(0, n_chunks)
    def _chunk(j):
      bi = j % NBUF
      o = j % NO
      wait_gather()

      @pl.when(j + HALF < n_chunks)
      def _():
        fire_gather(j + HALF)

      @pl.when(j >= NO)
      def _():
        wait_put()

      @pl.loop(0, CHUNK, unroll=4)
      def _row(r):
        for c in range(D_MODEL // LANES):
          sl = pl.ds(c * LANES, LANES)
          ot_v[o, r, sl] = rows_v[bi, r, sl] * SCALE

      pltpu.async_copy(
          ot_v.at[o], out_hbm.at[pl.ds(base + j * CHUNK, CHUNK)], psem
      )

    @pl.loop(0, NO)
    def _drain(_):
      wait_put()

  return emb_kernel


def kernel(x, table):
  b, s = x.shape
  n_idx = b * s
  idx = x.reshape(NW, n_idx // (NW * CHUNK), CHUNK).astype(jnp.int32)
  tblp = jnp.pad(table, ((0, 0), (0, D_MODEL)))
  out = _make_kernel(n_idx)(idx, tblp)
  return out.reshape(b, s, D_MODEL)


# speedup vs baseline: 1.5875x; 1.2155x over previous
import functools

import jax
import jax.numpy as jnp
from jax import lax
from jax.experimental import pallas as pl
from jax.experimental.pallas import tpu as pltpu
from jax.experimental.pallas import tpu_sc as plsc

D_MODEL = 64
SCALE = 8.0

_INFO = plsc.get_sparse_core_info()
NC = _INFO.num_cores
NS = _INFO.num_subcores
NW = NC * NS
LANES = _INFO.num_lanes

CHUNK = 128
NBUF = 8
HALF = NBUF // 2


def _make_kernel(n_idx: int):
  assert n_idx % (NW * CHUNK) == 0
  per_w = n_idx // NW
  n_chunks = per_w // CHUNK
  assert n_chunks > NBUF

  mesh = plsc.VectorSubcoreMesh(core_axis_name="c", subcore_axis_name="s")

  @functools.partial(
      pl.kernel,
      out_type=jax.ShapeDtypeStruct((n_idx, D_MODEL), jnp.float32),
      mesh=mesh,
      scratch_types=[
          pltpu.VMEM((n_chunks, CHUNK), jnp.int32),
          pltpu.VMEM((NBUF, CHUNK, D_MODEL), jnp.float32),
          pltpu.SemaphoreType.DMA,
          pltpu.SemaphoreType.DMA,
      ],
      compiler_params=pltpu.CompilerParams(use_tc_tiling_on_sc=False),
  )
  def emb_kernel(idx_hbm, table_hbm, out_hbm, idx_v, rows_v, gsem, psem):
    wid = lax.axis_index("s") * NC + lax.axis_index("c")
    base = wid * per_w
    pltpu.sync_copy(idx_hbm.at[wid], idx_v)

    def fire_gather(j):
      pltpu.async_copy(table_hbm.at[idx_v.at[j]], rows_v.at[j % NBUF], gsem)

    def wait_one(sem):
      pltpu.make_async_copy(rows_v.at[0], out_hbm.at[pl.ds(0, CHUNK)],
                            sem).wait()

    for j in range(HALF):
      fire_gather(j)

    @pl.loop(0, n_chunks)
    def _chunk(j):
      bi = j % NBUF
      wait_one(gsem)

      @pl.loop(0, CHUNK, unroll=4)
      def _row(r):
        for c in range(D_MODEL // LANES):
          sl = pl.ds(c * LANES, LANES)
          rows_v[bi, r, sl] = rows_v[bi, r, sl] * SCALE

      pltpu.async_copy(
          rows_v.at[bi], out_hbm.at[pl.ds(base + j * CHUNK, CHUNK)], psem
      )

      jn = j + HALF

      @pl.when(jn < n_chunks)
      def _():
        @pl.when(j >= HALF)
        def _():
          wait_one(psem)
        fire_gather(jn)

    @pl.loop(0, NBUF)
    def _drain(_):
      wait_one(psem)

  return emb_kernel


def kernel(x, table):
  b, s = x.shape
  n_idx = b * s
  idx = x.reshape(NW, n_idx // (NW * CHUNK), CHUNK).astype(jnp.int32)
  out = _make_kernel(n_idx)(idx, table)
  return out.reshape(b, s, D_MODEL)
